# Initial kernel scaffold; baseline (speedup 1.0000x reference)
import jax
import jax.numpy as jnp
from jax.experimental import pallas as pl


def kernel(node_feat, row1, col1, val1, row2, col2, val2, W1, b1, bn1_gamma, bn1_beta, strc_W, strc_gamma, strc_beta, policy, Wf, bf):
    raise NotImplementedError("write your pallas kernel here")


# speedup vs baseline: 4.0867x; 4.0867x over previous
import functools

import jax
import jax.numpy as jnp
from jax import lax
from jax.experimental import pallas as pl
from jax.experimental.pallas import tpu as pltpu
from jax.experimental.pallas import tpu_sc as plsc

N = 10000
IN_C = 128
HID = 64
OUT_C = 64
LAST_DIM = 448
POWER = 4

GROUP = 160
NGROUPS = 64
NPAD = GROUP * NGROUPS
NWORKERS = 32
ROW_BLK = 1280


@functools.lru_cache(maxsize=None)
def _make_spmm(c_dim: int, ec: int):
    mesh = plsc.VectorSubcoreMesh(core_axis_name="c", subcore_axis_name="s")
    ncols16 = c_dim // 16

    @functools.partial(
        pl.kernel,
        mesh=mesh,
        compiler_params=pltpu.CompilerParams(use_tc_tiling_on_sc=False),
        out_type=jax.ShapeDtypeStruct((NPAD, c_dim), jnp.float32),
        scratch_types=[
            pltpu.VMEM((ec,), jnp.int32),
            pltpu.VMEM((ec,), jnp.int32),
            pltpu.VMEM((ec,), jnp.float32),
            pltpu.VMEM((ec,), jnp.int32),
            pltpu.VMEM((ec, c_dim), jnp.float32),
            pltpu.VMEM((GROUP, c_dim), jnp.float32),
            pltpu.VMEM((80,), jnp.int32),
            pltpu.SemaphoreType.DMA,
            pltpu.SemaphoreType.DMA,
            pltpu.SemaphoreType.DMA,
            pltpu.SemaphoreType.DMA,
        ],
    )
    def spmm(x_hbm, col_hbm, row_hbm, val_hbm, offs_hbm, out_hbm,
             colv, rowv, valv, roffv, gbuf, acc, offsv,
             sem_c, sem_r, sem_v, sem_g):
        wid = lax.axis_index("s") * 2 + lax.axis_index("c")
        pltpu.sync_copy(offs_hbm, offsv)
        zero16 = jnp.zeros((16,), jnp.float32)

        for sub in range(2):
            g = wid * 2 + sub
            r0 = pl.multiple_of(g * GROUP, 8)
            off16 = offsv[pl.ds(g, 16)]
            ge0 = off16[0]
            ge1 = off16[1]
            base0 = (ge0 // 16) * 16
            nch = (ge1 - base0 + (ec - 1)) // ec

            def zrow(r, _):
                for j in range(ncols16):
                    acc[r, pl.ds(j * 16, 16)] = zero16
                return 0
            lax.fori_loop(0, GROUP, zrow, 0)

            def chunk(k, _):
                b = pl.multiple_of(base0 + k * ec, 16)
                cc = pltpu.async_copy(col_hbm.at[pl.ds(b, ec)], colv, sem_c)
                cr = pltpu.async_copy(row_hbm.at[pl.ds(b, ec)], rowv, sem_r)
                cv = pltpu.async_copy(val_hbm.at[pl.ds(b, ec)], valv, sem_v)
                cc.wait()
                cg = pltpu.async_copy(x_hbm.at[colv], gbuf, sem_g)
                cr.wait()
                cv.wait()
                for i in range(ec // 16):
                    eidx = (b + i * 16) + lax.iota(jnp.int32, 16)
                    ok = (eidx >= ge0) & (eidx < ge1)
                    v = jnp.where(ok, valv[pl.ds(i * 16, 16)], 0.0)
                    ro = jnp.clip(rowv[pl.ds(i * 16, 16)] - r0, 0, GROUP - 1)
                    valv[pl.ds(i * 16, 16)] = v
                    roffv[pl.ds(i * 16, 16)] = ro
                cg.wait()

                def edge16(q, _):
                    e0 = q * 16
                    ro16 = roffv[pl.ds(e0, 16)]
                    v16 = valv[pl.ds(e0, 16)]
                    for t in range(16):
                        ro = ro16[t]
                        v = v16[t]
                        for j in range(ncols16):
                            plsc.addupdate(
                                acc.at[ro, pl.ds(j * 16, 16)],
                                v * gbuf[e0 + t, pl.ds(j * 16, 16)])
                    return 0
                lax.fori_loop(0, ec // 16, edge16, 0)
                return 0
            lax.fori_loop(0, nch, chunk, 0)

            pltpu.sync_copy(acc, out_hbm.at[pl.ds(r0, GROUP)])

    return spmm


def _spmm64(*args):
    return _make_spmm(64, 128)(*args)


def _spmm128(*args):
    return _make_spmm(128, 128)(*args)


def _spmm448(*args):
    return _make_spmm(448, 32)(*args)


def _prep_edges(row, col, val):
    e = row.shape[0]
    pe = ((e + 255) // 128) * 128
    pad = pe - e
    col32 = col.astype(jnp.int32)
    row32 = row.astype(jnp.int32)
    fill = (jnp.arange(pad, dtype=jnp.int32) * 97) % N
    colp = jnp.concatenate([col32, fill])
    rowp = jnp.concatenate([row32, jnp.zeros((pad,), jnp.int32)])
    valp = jnp.concatenate([val, jnp.zeros((pad,), jnp.float32)])
    bounds = jnp.arange(NGROUPS + 1, dtype=jnp.int32) * GROUP
    offs = jnp.searchsorted(row32, bounds).astype(jnp.int32)
    offsp = jnp.concatenate([offs, jnp.full((80 - (NGROUPS + 1),), e, jnp.int32)])
    return colp, rowp, valp, offsp


def _embed_body(nf_ref, w_ref, b_ref, o_ref):
    z = jnp.dot(nf_ref[...], w_ref[...], preferred_element_type=jnp.float32)
    o_ref[...] = jnp.maximum(z + b_ref[...], 0.0)


def _embed(node_feat, w1, b1):
    return pl.pallas_call(
        _embed_body,
        grid=(25,),
        in_specs=[
            pl.BlockSpec((400, IN_C), lambda i: (i, 0)),
            pl.BlockSpec((IN_C, HID), lambda i: (0, 0)),
            pl.BlockSpec((1, HID), lambda i: (0, 0)),
        ],
        out_specs=pl.BlockSpec((400, HID), lambda i: (i, 0)),
        out_shape=jax.ShapeDtypeStruct((N, HID), jnp.float32),
    )(node_feat, w1, b1.reshape(1, HID))


def _stats_body(y_ref, o_ref):
    i = pl.program_id(0)

    @pl.when(i == 0)
    def _():
        o_ref[...] = jnp.zeros_like(o_ref)

    blk = y_ref[...]
    s1 = jnp.sum(blk, axis=0, keepdims=True)
    s2 = jnp.sum(blk * blk, axis=0, keepdims=True)
    c = blk.shape[1]
    upd = jnp.concatenate([s1, s2, jnp.zeros((6, c), jnp.float32)], axis=0)
    o_ref[...] = o_ref[...] + upd


def _stats(y):
    c = y.shape[1]
    return pl.pallas_call(
        _stats_body,
        grid=(8,),
        in_specs=[pl.BlockSpec((ROW_BLK, c), lambda i: (i, 0))],
        out_specs=pl.BlockSpec((8, c), lambda i: (0, 0)),
        out_shape=jax.ShapeDtypeStruct((8, c), jnp.float32),
    )(y)


def _bn_body(y_ref, st_ref, g_ref, b_ref, o_ref):
    inv_n = jnp.float32(1.0 / N)
    mu = st_ref[0:1, :] * inv_n
    var = st_ref[1:2, :] * inv_n - mu * mu
    scale = lax.rsqrt(var + 1e-5) * g_ref[...]
    o_ref[...] = (y_ref[...] - mu) * scale + b_ref[...]


def _bn_apply(y, st, gamma, beta):
    c = y.shape[1]
    return pl.pallas_call(
        _bn_body,
        grid=(8,),
        in_specs=[
            pl.BlockSpec((ROW_BLK, c), lambda i: (i, 0)),
            pl.BlockSpec((8, c), lambda i: (0, 0)),
            pl.BlockSpec((1, c), lambda i: (0, 0)),
            pl.BlockSpec((1, c), lambda i: (0, 0)),
        ],
        out_specs=pl.BlockSpec((ROW_BLK, c), lambda i: (i, 0)),
        out_shape=jax.ShapeDtypeStruct((NPAD, c), jnp.float32),
    )(y, st, gamma.reshape(1, c), beta.reshape(1, c))


def _bn_acc_body(y_ref, st_ref, g_ref, b_ref, acc_ref, o_ref, accout_ref):
    inv_n = jnp.float32(1.0 / N)
    mu = st_ref[0:1, :] * inv_n
    var = st_ref[1:2, :] * inv_n - mu * mu
    scale = lax.rsqrt(var + 1e-5) * g_ref[...]
    z = (y_ref[...] - mu) * scale + b_ref[...]
    o_ref[...] = z
    accout_ref[...] = acc_ref[...] + z


def _bn_apply_acc(y, st, gamma, beta, acc):
    c = y.shape[1]
    return pl.pallas_call(
        _bn_acc_body,
        grid=(8,),
        in_specs=[
            pl.BlockSpec((ROW_BLK, c), lambda i: (i, 0)),
            pl.BlockSpec((8, c), lambda i: (0, 0)),
            pl.BlockSpec((1, c), lambda i: (0, 0)),
            pl.BlockSpec((1, c), lambda i: (0, 0)),
            pl.BlockSpec((ROW_BLK, c), lambda i: (i, 0)),
        ],
        out_specs=[
            pl.BlockSpec((ROW_BLK, c), lambda i: (i, 0)),
            pl.BlockSpec((ROW_BLK, c), lambda i: (i, 0)),
        ],
        out_shape=[
            jax.ShapeDtypeStruct((NPAD, c), jnp.float32),
            jax.ShapeDtypeStruct((NPAD, c), jnp.float32),
        ],
        input_output_aliases={4: 1},
    )(y, st, gamma.reshape(1, c), beta.reshape(1, c), acc)


def _final_body(xc_ref, xa_ref, pol_ref, wf_ref, bf_ref, o_ref):
    p = pol_ref[...]
    e = jnp.exp(p - jnp.max(p))
    pp = e / jnp.sum(e)
    w_cat = pp[0:1, 0:1]
    w_adj = pp[0:1, 1:2] * jnp.float32(1.0 / POWER)
    x = w_cat * xc_ref[...] + w_adj * xa_ref[...]
    z = jnp.dot(x, wf_ref[...], preferred_element_type=jnp.float32)
    o_ref[...] = z + bf_ref[...]


def _final(xcat, xadj, policy, wf, bf):
    return pl.pallas_call(
        _final_body,
        grid=(8,),
        in_specs=[
            pl.BlockSpec((ROW_BLK, LAST_DIM), lambda i: (i, 0)),
            pl.BlockSpec((ROW_BLK, LAST_DIM), lambda i: (i, 0)),
            pl.BlockSpec((1, 2), lambda i: (0, 0)),
            pl.BlockSpec((LAST_DIM, OUT_C), lambda i: (0, 0)),
            pl.BlockSpec((1, OUT_C), lambda i: (0, 0)),
        ],
        out_specs=pl.BlockSpec((ROW_BLK, OUT_C), lambda i: (i, 0)),
        out_shape=jax.ShapeDtypeStruct((NPAD, OUT_C), jnp.float32),
    )(xcat, xadj, policy.reshape(1, 2), wf, bf.reshape(1, OUT_C))


def kernel(node_feat, row1, col1, val1, row2, col2, val2, W1, b1,
           bn1_gamma, bn1_beta, strc_W, strc_gamma, strc_beta, policy,
           Wf, bf):
    c1p, r1p, v1p, o1p = _prep_edges(row1, col1, val1)
    c2p, r2p, v2p, o2p = _prep_edges(row2, col2, val2)

    pad_rows = NPAD - N
    x0 = _embed(node_feat, W1, b1)
    x0p = jnp.concatenate(
        [x0, jnp.zeros((pad_rows, HID), jnp.float32)], axis=0)

    a1 = _spmm64(x0p, c1p, r1p, v1p, o1p)
    a2 = _spmm64(x0p, c2p, r2p, v2p, o2p)
    y1 = jnp.concatenate([a1, a2], axis=1)
    x1 = _bn_apply(y1, _stats(y1), bn1_gamma, bn1_beta)

    b1_ = _spmm128(x1, c1p, r1p, v1p, o1p)
    b2_ = _spmm128(x1, c2p, r2p, v2p, o2p)
    x2 = jnp.concatenate([b1_, b2_], axis=1)

    xcat = jnp.concatenate([x0p, x1, x2], axis=1)

    w = jnp.concatenate(
        [strc_W, jnp.zeros((pad_rows, LAST_DIM), jnp.float32)], axis=0)
    xadj = jnp.zeros((NPAD, LAST_DIM), jnp.float32)
    for i in range(POWER):
        t = _spmm448(w, c1p, r1p, v1p, o1p)
        w, xadj = _bn_apply_acc(t, _stats(t), strc_gamma[i], strc_beta[i],
                                xadj)

    out = _final(xcat, xadj, policy, Wf, bf)
    return out[:N]


# speedup vs baseline: 9.0867x; 2.2235x over previous
import functools

import jax
import jax.numpy as jnp
from jax import lax
from jax.experimental import pallas as pl
from jax.experimental.pallas import tpu as pltpu
from jax.experimental.pallas import tpu_sc as plsc

N = 10000
IN_C = 128
HID = 64
OUT_C = 64
LAST_DIM = 448
POWER = 4

GROUP = 160
NGROUPS = 64
NPAD = GROUP * NGROUPS
NWORKERS = 32
ROW_BLK = 1280


EC = 128


def _spmm_body(x_sh, colh, rowh, valh, offsv, out_hbm, colv, rowv, valv,
               gbuf, acc, sems, gsems, wid, c_p, c_out_off):
    nj = c_p // 16
    zero16 = jnp.zeros((16,), jnp.float32)

    for sub in range(2):
        g = wid * 2 + sub
        r0 = pl.multiple_of(g * GROUP, 8)
        off16 = offsv[pl.ds(g, 16)]
        ge0 = off16[0]
        ge1 = off16[1]
        base0 = (ge0 // 16) * 16
        nch = (ge1 - base0 + (EC - 1)) // EC

        def zrow(r, _):
            for j in range(nj):
                acc[r, pl.ds(j * 16, 16)] = zero16
            return 0
        lax.fori_loop(0, GROUP, zrow, 0)

        def stage(k, soff):
            b = pl.multiple_of(base0 + k * EC, 16)
            pltpu.async_copy(colh.at[pl.ds(b, EC)],
                             colv.at[pl.ds(soff, EC)], sems[0])
            pltpu.async_copy(rowh.at[pl.ds(b, EC)],
                             rowv.at[pl.ds(soff, EC)], sems[1])
            pltpu.async_copy(valh.at[pl.ds(b, EC)],
                             valv.at[pl.ds(soff, EC)], sems[2])

        def wait_stage(soff):
            for i in range(3):
                pltpu.make_async_copy(colh.at[pl.ds(0, EC)],
                                      colv.at[pl.ds(soff, EC)],
                                      sems[i]).wait()

        stage(0, 0)

        def chunk(k, _):
            soff = pl.multiple_of(lax.rem(k, 2) * EC, EC)
            b = pl.multiple_of(base0 + k * EC, 16)
            wait_stage(soff)
            cg = pltpu.async_copy(x_sh.at[colv.at[pl.ds(soff, EC)]],
                                  gbuf.at[pl.ds(soff, EC)], gsems)
            stage(k + 1, EC - soff)
            cg.wait()

            def edge16(q, _):
                e0 = soff + q * 16
                eidx = (b - soff + e0) + lax.iota(jnp.int32, 16)
                ok = (eidx >= ge0) & (eidx < ge1)
                v16 = jnp.where(ok, valv[pl.ds(e0, 16)], 0.0)
                ro16 = jnp.clip(rowv[pl.ds(e0, 16)] - r0, 0, GROUP - 1)
                prow = ro16[0]
                rg = [None] * nj
                for t in range(16):
                    ro = ro16[t]
                    v = v16[t]
                    if t == 0:
                        for j in range(nj):
                            rg[j] = v * gbuf[e0, pl.ds(j * 16, 16)]
                        continue
                    flush = ro != prow

                    @pl.when(flush)
                    def _(prow=prow, rg=tuple(rg)):
                        for j in range(nj):
                            plsc.addupdate(acc.at[prow, pl.ds(j * 16, 16)],
                                           rg[j])
                    for j in range(nj):
                        contrib = v * gbuf[e0 + t, pl.ds(j * 16, 16)]
                        rg[j] = jnp.where(flush, contrib, rg[j] + contrib)
                    prow = ro
                for j in range(nj):
                    plsc.addupdate(acc.at[prow, pl.ds(j * 16, 16)], rg[j])
                return 0
            lax.fori_loop(0, EC // 16, edge16, 0)
            return 0

        lax.fori_loop(0, nch, chunk, 0)
        wait_stage(pl.multiple_of(lax.rem(nch, 2) * EC, EC))

        pltpu.sync_copy(
            acc, out_hbm.at[pl.ds(r0, GROUP), pl.ds(c_out_off, c_p)])


@functools.lru_cache(maxsize=None)
def _make_spmm_kernel(c_src: int, p_w: int, nsets: int):
    mesh = plsc.VectorSubcoreMesh(core_axis_name="c", subcore_axis_name="s")
    npan = c_src // p_w

    @functools.partial(
        pl.kernel,
        mesh=mesh,
        compiler_params=pltpu.CompilerParams(use_tc_tiling_on_sc=False),
        out_type=jax.ShapeDtypeStruct((NPAD, nsets * c_src), jnp.float32),
        scratch_types=[
            pltpu.VMEM_SHARED((NPAD, p_w), jnp.float32),
            pltpu.VMEM((2 * EC,), jnp.int32),
            pltpu.VMEM((2 * EC,), jnp.int32),
            pltpu.VMEM((2 * EC,), jnp.float32),
            pltpu.VMEM((2 * EC, p_w), jnp.float32),
            pltpu.VMEM((GROUP, p_w), jnp.float32),
        ] + [pltpu.VMEM((80,), jnp.int32)] * nsets + [
            pltpu.SemaphoreType.DMA,
            pltpu.SemaphoreType.DMA,
            pltpu.SemaphoreType.DMA,
            pltpu.SemaphoreType.DMA,
            pltpu.SemaphoreType.DMA,
        ],
    )
    def spmm(*refs):
        x_hbm = refs[0]
        esets = [refs[1 + 4 * i:5 + 4 * i] for i in range(nsets)]
        out_hbm = refs[1 + 4 * nsets]
        sc = 2 + 4 * nsets
        x_sh, colv, rowv, valv, gbuf, acc = refs[sc:sc + 6]
        offsv = refs[sc + 6:sc + 6 + nsets]
        s0, s1, s2, sg, sx = refs[sc + 6 + nsets:sc + 11 + nsets]
        sems = (s0, s1, s2)

        sid = lax.axis_index("s")
        wid = sid * 2 + lax.axis_index("c")
        for i in range(nsets):
            pltpu.sync_copy(esets[i][3], offsv[i])
        for p in range(npan):
            plsc.subcore_barrier()

            @pl.when(sid == 0)
            def _(p=p):
                pltpu.async_copy(
                    x_hbm.at[:, pl.ds(p * p_w, p_w)], x_sh, sx).wait()
            plsc.subcore_barrier()
            for i in range(nsets):
                ch, rh, vh, _oh = esets[i]
                _spmm_body(x_sh, ch, rh, vh, offsv[i], out_hbm, colv, rowv,
                           valv, gbuf, acc, sems, sg, wid, p_w,
                           i * c_src + p * p_w)

    return spmm


def _conv_spmm(c_dim, x, e1, e2):
    return _make_spmm_kernel(c_dim, 64, 2)(x, *e1, *e2)


def _strc_spmm(x, e1):
    return _make_spmm_kernel(LAST_DIM, 112, 1)(x, *e1)


def _prep_edges(row, col, val):
    e = row.shape[0]
    pe = ((e + 511) // 128) * 128
    pad = pe - e
    col32 = col.astype(jnp.int32)
    row32 = row.astype(jnp.int32)
    fill = (jnp.arange(pad, dtype=jnp.int32) * 97) % N
    colp = jnp.concatenate([col32, fill])
    rowp = jnp.concatenate([row32, jnp.zeros((pad,), jnp.int32)])
    valp = jnp.concatenate([val, jnp.zeros((pad,), jnp.float32)])
    bounds = jnp.arange(NGROUPS + 1, dtype=jnp.int32) * GROUP
    offs = jnp.searchsorted(row32, bounds).astype(jnp.int32)
    offsp = jnp.concatenate([offs, jnp.full((80 - (NGROUPS + 1),), e, jnp.int32)])
    return colp, rowp, valp, offsp


def _embed_body(nf_ref, w_ref, b_ref, o_ref):
    z = jnp.dot(nf_ref[...], w_ref[...], preferred_element_type=jnp.float32)
    o_ref[...] = jnp.maximum(z + b_ref[...], 0.0)


def _embed(node_feat, w1, b1):
    return pl.pallas_call(
        _embed_body,
        grid=(25,),
        in_specs=[
            pl.BlockSpec((400, IN_C), lambda i: (i, 0)),
            pl.BlockSpec((IN_C, HID), lambda i: (0, 0)),
            pl.BlockSpec((1, HID), lambda i: (0, 0)),
        ],
        out_specs=pl.BlockSpec((400, HID), lambda i: (i, 0)),
        out_shape=jax.ShapeDtypeStruct((N, HID), jnp.float32),
    )(node_feat, w1, b1.reshape(1, HID))


def _stats_body(y_ref, o_ref):
    i = pl.program_id(0)

    @pl.when(i == 0)
    def _():
        o_ref[...] = jnp.zeros_like(o_ref)

    blk = y_ref[...]
    s1 = jnp.sum(blk, axis=0, keepdims=True)
    s2 = jnp.sum(blk * blk, axis=0, keepdims=True)
    c = blk.shape[1]
    upd = jnp.concatenate([s1, s2, jnp.zeros((6, c), jnp.float32)], axis=0)
    o_ref[...] = o_ref[...] + upd


def _stats(y):
    c = y.shape[1]
    return pl.pallas_call(
        _stats_body,
        grid=(8,),
        in_specs=[pl.BlockSpec((ROW_BLK, c), lambda i: (i, 0))],
        out_specs=pl.BlockSpec((8, c), lambda i: (0, 0)),
        out_shape=jax.ShapeDtypeStruct((8, c), jnp.float32),
    )(y)


def _bn_body(y_ref, st_ref, g_ref, b_ref, o_ref):
    inv_n = jnp.float32(1.0 / N)
    mu = st_ref[0:1, :] * inv_n
    var = st_ref[1:2, :] * inv_n - mu * mu
    scale = lax.rsqrt(var + 1e-5) * g_ref[...]
    o_ref[...] = (y_ref[...] - mu) * scale + b_ref[...]


def _bn_apply(y, st, gamma, beta):
    c = y.shape[1]
    return pl.pallas_call(
        _bn_body,
        grid=(8,),
        in_specs=[
            pl.BlockSpec((ROW_BLK, c), lambda i: (i, 0)),
            pl.BlockSpec((8, c), lambda i: (0, 0)),
            pl.BlockSpec((1, c), lambda i: (0, 0)),
            pl.BlockSpec((1, c), lambda i: (0, 0)),
        ],
        out_specs=pl.BlockSpec((ROW_BLK, c), lambda i: (i, 0)),
        out_shape=jax.ShapeDtypeStruct((NPAD, c), jnp.float32),
    )(y, st, gamma.reshape(1, c), beta.reshape(1, c))


def _bn_acc_body(y_ref, st_ref, g_ref, b_ref, acc_ref, o_ref, accout_ref):
    inv_n = jnp.float32(1.0 / N)
    mu = st_ref[0:1, :] * inv_n
    var = st_ref[1:2, :] * inv_n - mu * mu
    scale = lax.rsqrt(var + 1e-5) * g_ref[...]
    z = (y_ref[...] - mu) * scale + b_ref[...]
    o_ref[...] = z
    accout_ref[...] = acc_ref[...] + z


def _bn_apply_acc(y, st, gamma, beta, acc):
    c = y.shape[1]
    return pl.pallas_call(
        _bn_acc_body,
        grid=(8,),
        in_specs=[
            pl.BlockSpec((ROW_BLK, c), lambda i: (i, 0)),
            pl.BlockSpec((8, c), lambda i: (0, 0)),
            pl.BlockSpec((1, c), lambda i: (0, 0)),
            pl.BlockSpec((1, c), lambda i: (0, 0)),
            pl.BlockSpec((ROW_BLK, c), lambda i: (i, 0)),
        ],
        out_specs=[
            pl.BlockSpec((ROW_BLK, c), lambda i: (i, 0)),
            pl.BlockSpec((ROW_BLK, c), lambda i: (i, 0)),
        ],
        out_shape=[
            jax.ShapeDtypeStruct((NPAD, c), jnp.float32),
            jax.ShapeDtypeStruct((NPAD, c), jnp.float32),
        ],
        input_output_aliases={4: 1},
    )(y, st, gamma.reshape(1, c), beta.reshape(1, c), acc)


def _final_body(xc_ref, xa_ref, pol_ref, wf_ref, bf_ref, o_ref):
    p = pol_ref[...]
    e = jnp.exp(p - jnp.max(p))
    pp = e / jnp.sum(e)
    w_cat = pp[0:1, 0:1]
    w_adj = pp[0:1, 1:2] * jnp.float32(1.0 / POWER)
    x = w_cat * xc_ref[...] + w_adj * xa_ref[...]
    z = jnp.dot(x, wf_ref[...], preferred_element_type=jnp.float32)
    o_ref[...] = z + bf_ref[...]


def _final(xcat, xadj, policy, wf, bf):
    return pl.pallas_call(
        _final_body,
        grid=(8,),
        in_specs=[
            pl.BlockSpec((ROW_BLK, LAST_DIM), lambda i: (i, 0)),
            pl.BlockSpec((ROW_BLK, LAST_DIM), lambda i: (i, 0)),
            pl.BlockSpec((1, 2), lambda i: (0, 0)),
            pl.BlockSpec((LAST_DIM, OUT_C), lambda i: (0, 0)),
            pl.BlockSpec((1, OUT_C), lambda i: (0, 0)),
        ],
        out_specs=pl.BlockSpec((ROW_BLK, OUT_C), lambda i: (i, 0)),
        out_shape=jax.ShapeDtypeStruct((NPAD, OUT_C), jnp.float32),
    )(xcat, xadj, policy.reshape(1, 2), wf, bf.reshape(1, OUT_C))


def kernel(node_feat, row1, col1, val1, row2, col2, val2, W1, b1,
           bn1_gamma, bn1_beta, strc_W, strc_gamma, strc_beta, policy,
           Wf, bf):
    c1p, r1p, v1p, o1p = _prep_edges(row1, col1, val1)
    c2p, r2p, v2p, o2p = _prep_edges(row2, col2, val2)

    pad_rows = NPAD - N
    x0 = _embed(node_feat, W1, b1)
    x0p = jnp.concatenate(
        [x0, jnp.zeros((pad_rows, HID), jnp.float32)], axis=0)

    e1 = (c1p, r1p, v1p, o1p)
    e2 = (c2p, r2p, v2p, o2p)

    y1 = _conv_spmm(HID, x0p, e1, e2)
    x1 = _bn_apply(y1, _stats(y1), bn1_gamma, bn1_beta)

    x2 = _conv_spmm(2 * HID, x1, e1, e2)

    xcat = jnp.concatenate([x0p, x1, x2], axis=1)

    w = jnp.concatenate(
        [strc_W, jnp.zeros((pad_rows, LAST_DIM), jnp.float32)], axis=0)
    xadj = jnp.zeros((NPAD, LAST_DIM), jnp.float32)
    for i in range(POWER):
        t = _strc_spmm(w, e1)
        w, xadj = _bn_apply_acc(t, _stats(t), strc_gamma[i], strc_beta[i],
                                xadj)

    out = _final(xcat, xadj, policy, Wf, bf)
    return out[:N]


# speedup vs baseline: 16.0668x; 1.7682x over previous
import functools

import jax
import jax.numpy as jnp
from jax import lax
from jax.experimental import pallas as pl
from jax.experimental.pallas import tpu as pltpu
from jax.experimental.pallas import tpu_sc as plsc

N = 10000
IN_C = 128
HID = 64
OUT_C = 64
LAST_DIM = 448
POWER = 4

GROUP = 160
NGROUPS = 64
NPAD = GROUP * NGROUPS
NWORKERS = 32
ROW_BLK = 1280


EC = 128


def _spmm_body(x_sh, colh, rowh, valh, offsv, out_hbm, colv, rowv, valv,
               gbuf, acc, sems, gsems, wid, c_p, c_out_off):
    nj = c_p // 16
    zero16 = jnp.zeros((16,), jnp.float32)

    for sub in range(2):
        g = wid * 2 + sub
        r0 = pl.multiple_of(g * GROUP, 8)
        off16 = offsv[pl.ds(g, 16)]
        ge0 = off16[0]
        ge1 = off16[1]
        base0 = (ge0 // 16) * 16
        nch = (ge1 - base0 + (EC - 1)) // EC
        nch2 = (nch + 1) // 2

        def zrow(r, _):
            for j in range(nj):
                acc[r, pl.ds(j * 16, 16)] = zero16
            return 0
        lax.fori_loop(0, GROUP, zrow, 0)

        def stage_col(k, s):
            b = pl.multiple_of(base0 + k * EC, 16)
            pltpu.async_copy(colh.at[pl.ds(b, EC)],
                             colv.at[pl.ds(s * EC, EC)], sems[s])

        def wait_col(s):
            pltpu.make_async_copy(colh.at[pl.ds(0, EC)],
                                  colv.at[pl.ds(s * EC, EC)],
                                  sems[s]).wait()

        def stage_vr(k, s):
            b = pl.multiple_of(base0 + k * EC, 16)
            sem = sems[2 + s]
            pltpu.async_copy(rowh.at[pl.ds(b, EC)],
                             rowv.at[pl.ds(s * EC, EC)], sem)
            pltpu.async_copy(valh.at[pl.ds(b, EC)],
                             valv.at[pl.ds(s * EC, EC)], sem)

        def wait_vr(s):
            sem = sems[2 + s]
            for _ in range(2):
                pltpu.make_async_copy(rowh.at[pl.ds(0, EC)],
                                      rowv.at[pl.ds(s * EC, EC)],
                                      sem).wait()

        def fire_gather(s):
            pltpu.async_copy(x_sh.at[colv.at[pl.ds(s * EC, EC)]],
                             gbuf.at[pl.ds(s * EC, EC)], gsems)

        def wait_gather(s):
            pltpu.make_async_copy(x_sh.at[colv.at[pl.ds(s * EC, EC)]],
                                  gbuf.at[pl.ds(s * EC, EC)], gsems).wait()

        stage_col(0, 0)
        stage_col(1, 1)
        stage_vr(0, 0)
        wait_col(0)
        fire_gather(0)

        def chunk_s(kk, s):
            k = 2 * kk + s
            soff = s * EC
            b = pl.multiple_of(base0 + k * EC, 16)
            wait_gather(s)
            wait_col(1 - s)
            fire_gather(1 - s)
            stage_col(k + 2, s)
            stage_vr(k + 1, 1 - s)
            wait_vr(s)

            def edge16(q, _):
                e0 = soff + q * 16
                eidx = (b - soff + e0) + lax.iota(jnp.int32, 16)
                ok = (eidx >= ge0) & (eidx < ge1)
                v16 = jnp.where(ok, valv[pl.ds(e0, 16)], 0.0)
                ro16 = jnp.clip(rowv[pl.ds(e0, 16)] - r0, 0, GROUP - 1)
                single = ro16[0] == ro16[15]

                @pl.when(single)
                def _():
                    rg = [v16[0] * gbuf[e0, pl.ds(j * 16, 16)]
                          for j in range(nj)]
                    for t in range(1, 16):
                        v = v16[t]
                        for j in range(nj):
                            rg[j] = rg[j] + v * gbuf[e0 + t,
                                                     pl.ds(j * 16, 16)]
                    for j in range(nj):
                        plsc.addupdate(acc.at[ro16[0], pl.ds(j * 16, 16)],
                                       rg[j])

                @pl.when(jnp.logical_not(single))
                def _():
                    prow = ro16[0]
                    rg = [None] * nj
                    for t in range(16):
                        ro = ro16[t]
                        v = v16[t]
                        if t == 0:
                            for j in range(nj):
                                rg[j] = v * gbuf[e0, pl.ds(j * 16, 16)]
                            continue
                        flush = ro != prow

                        @pl.when(flush)
                        def _(prow=prow, rg=tuple(rg)):
                            for j in range(nj):
                                plsc.addupdate(
                                    acc.at[prow, pl.ds(j * 16, 16)], rg[j])
                        for j in range(nj):
                            contrib = v * gbuf[e0 + t, pl.ds(j * 16, 16)]
                            rg[j] = jnp.where(flush, contrib,
                                              rg[j] + contrib)
                        prow = ro
                    for j in range(nj):
                        plsc.addupdate(acc.at[prow, pl.ds(j * 16, 16)],
                                       rg[j])
                return 0
            lax.fori_loop(0, EC // 16, edge16, 0)

        def chunk_pair(kk, _):
            chunk_s(kk, 0)
            chunk_s(kk, 1)
            return 0

        lax.fori_loop(0, nch2, chunk_pair, 0)
        wait_gather(0)
        wait_col(1)
        wait_vr(0)

        pltpu.sync_copy(
            acc, out_hbm.at[pl.ds(r0, GROUP), pl.ds(c_out_off, c_p)])


@functools.lru_cache(maxsize=None)
def _make_spmm_kernel(c_src: int, p_w: int, nsets: int):
    mesh = plsc.VectorSubcoreMesh(core_axis_name="c", subcore_axis_name="s")
    npan = c_src // p_w

    @functools.partial(
        pl.kernel,
        mesh=mesh,
        compiler_params=pltpu.CompilerParams(use_tc_tiling_on_sc=False),
        out_type=jax.ShapeDtypeStruct((NPAD, nsets * c_src), jnp.float32),
        scratch_types=[
            pltpu.VMEM_SHARED((NPAD, p_w), jnp.float32),
            pltpu.VMEM((2 * EC,), jnp.int32),
            pltpu.VMEM((2 * EC,), jnp.int32),
            pltpu.VMEM((2 * EC,), jnp.float32),
            pltpu.VMEM((2 * EC, p_w), jnp.float32),
            pltpu.VMEM((GROUP, p_w), jnp.float32),
        ] + [pltpu.VMEM((80,), jnp.int32)] * nsets + [
            pltpu.SemaphoreType.DMA,
            pltpu.SemaphoreType.DMA,
            pltpu.SemaphoreType.DMA,
            pltpu.SemaphoreType.DMA,
            pltpu.SemaphoreType.DMA,
            pltpu.SemaphoreType.DMA,
        ],
    )
    def spmm(*refs):
        x_hbm = refs[0]
        esets = [refs[1 + 4 * i:5 + 4 * i] for i in range(nsets)]
        out_hbm = refs[1 + 4 * nsets]
        sc = 2 + 4 * nsets
        x_sh, colv, rowv, valv, gbuf, acc = refs[sc:sc + 6]
        offsv = refs[sc + 6:sc + 6 + nsets]
        s0, s1, s2, s3, sg, sx = refs[sc + 6 + nsets:sc + 12 + nsets]
        sems = (s0, s1, s2, s3)

        sid = lax.axis_index("s")
        wid = sid * 2 + lax.axis_index("c")
        for i in range(nsets):
            pltpu.sync_copy(esets[i][3], offsv[i])
        for p in range(npan):
            plsc.subcore_barrier()

            @pl.when(sid == 0)
            def _(p=p):
                pltpu.async_copy(
                    x_hbm.at[:, pl.ds(p * p_w, p_w)], x_sh, sx).wait()
            plsc.subcore_barrier()
            for i in range(nsets):
                ch, rh, vh, _oh = esets[i]
                _spmm_body(x_sh, ch, rh, vh, offsv[i], out_hbm, colv, rowv,
                           valv, gbuf, acc, sems, sg, wid, p_w,
                           i * c_src + p * p_w)

    return spmm


def _conv_spmm(c_dim, x, e1, e2):
    return _make_spmm_kernel(c_dim, 64, 2)(x, *e1, *e2)


def _strc_spmm(x, e1):
    return _make_spmm_kernel(LAST_DIM, 112, 1)(x, *e1)


def _prep_edges(row, col, val):
    e = row.shape[0]
    pe = ((e + 1023) // 128) * 128
    pad = pe - e
    col32 = col.astype(jnp.int32)
    row32 = row.astype(jnp.int32)
    fill = (jnp.arange(pad, dtype=jnp.int32) * 97) % N
    colp = jnp.concatenate([col32, fill])
    rowp = jnp.concatenate([row32, jnp.zeros((pad,), jnp.int32)])
    valp = jnp.concatenate([val, jnp.zeros((pad,), jnp.float32)])
    bounds = jnp.arange(NGROUPS + 1, dtype=jnp.int32) * GROUP
    offs = jnp.searchsorted(row32, bounds).astype(jnp.int32)
    offsp = jnp.concatenate([offs, jnp.full((80 - (NGROUPS + 1),), e, jnp.int32)])
    return colp, rowp, valp, offsp


def _embed_body(nf_ref, w_ref, b_ref, o_ref):
    z = jnp.dot(nf_ref[...], w_ref[...], preferred_element_type=jnp.float32)
    o_ref[...] = jnp.maximum(z + b_ref[...], 0.0)


def _embed(node_feat, w1, b1):
    return pl.pallas_call(
        _embed_body,
        grid=(25,),
        in_specs=[
            pl.BlockSpec((400, IN_C), lambda i: (i, 0)),
            pl.BlockSpec((IN_C, HID), lambda i: (0, 0)),
            pl.BlockSpec((1, HID), lambda i: (0, 0)),
        ],
        out_specs=pl.BlockSpec((400, HID), lambda i: (i, 0)),
        out_shape=jax.ShapeDtypeStruct((N, HID), jnp.float32),
    )(node_feat, w1, b1.reshape(1, HID))


def _stats_body(y_ref, o_ref):
    i = pl.program_id(0)

    @pl.when(i == 0)
    def _():
        o_ref[...] = jnp.zeros_like(o_ref)

    blk = y_ref[...]
    s1 = jnp.sum(blk, axis=0, keepdims=True)
    s2 = jnp.sum(blk * blk, axis=0, keepdims=True)
    c = blk.shape[1]
    upd = jnp.concatenate([s1, s2, jnp.zeros((6, c), jnp.float32)], axis=0)
    o_ref[...] = o_ref[...] + upd


def _stats(y):
    c = y.shape[1]
    return pl.pallas_call(
        _stats_body,
        grid=(8,),
        in_specs=[pl.BlockSpec((ROW_BLK, c), lambda i: (i, 0))],
        out_specs=pl.BlockSpec((8, c), lambda i: (0, 0)),
        out_shape=jax.ShapeDtypeStruct((8, c), jnp.float32),
    )(y)


def _bn_body(y_ref, st_ref, g_ref, b_ref, o_ref):
    inv_n = jnp.float32(1.0 / N)
    mu = st_ref[0:1, :] * inv_n
    var = st_ref[1:2, :] * inv_n - mu * mu
    scale = lax.rsqrt(var + 1e-5) * g_ref[...]
    o_ref[...] = (y_ref[...] - mu) * scale + b_ref[...]


def _bn_apply(y, st, gamma, beta):
    c = y.shape[1]
    return pl.pallas_call(
        _bn_body,
        grid=(8,),
        in_specs=[
            pl.BlockSpec((ROW_BLK, c), lambda i: (i, 0)),
            pl.BlockSpec((8, c), lambda i: (0, 0)),
            pl.BlockSpec((1, c), lambda i: (0, 0)),
            pl.BlockSpec((1, c), lambda i: (0, 0)),
        ],
        out_specs=pl.BlockSpec((ROW_BLK, c), lambda i: (i, 0)),
        out_shape=jax.ShapeDtypeStruct((NPAD, c), jnp.float32),
    )(y, st, gamma.reshape(1, c), beta.reshape(1, c))


def _bn_acc_body(y_ref, st_ref, g_ref, b_ref, acc_ref, o_ref, accout_ref):
    inv_n = jnp.float32(1.0 / N)
    mu = st_ref[0:1, :] * inv_n
    var = st_ref[1:2, :] * inv_n - mu * mu
    scale = lax.rsqrt(var + 1e-5) * g_ref[...]
    z = (y_ref[...] - mu) * scale + b_ref[...]
    o_ref[...] = z
    accout_ref[...] = acc_ref[...] + z


def _bn_apply_acc(y, st, gamma, beta, acc):
    c = y.shape[1]
    return pl.pallas_call(
        _bn_acc_body,
        grid=(8,),
        in_specs=[
            pl.BlockSpec((ROW_BLK, c), lambda i: (i, 0)),
            pl.BlockSpec((8, c), lambda i: (0, 0)),
            pl.BlockSpec((1, c), lambda i: (0, 0)),
            pl.BlockSpec((1, c), lambda i: (0, 0)),
            pl.BlockSpec((ROW_BLK, c), lambda i: (i, 0)),
        ],
        out_specs=[
            pl.BlockSpec((ROW_BLK, c), lambda i: (i, 0)),
            pl.BlockSpec((ROW_BLK, c), lambda i: (i, 0)),
        ],
        out_shape=[
            jax.ShapeDtypeStruct((NPAD, c), jnp.float32),
            jax.ShapeDtypeStruct((NPAD, c), jnp.float32),
        ],
        input_output_aliases={4: 1},
    )(y, st, gamma.reshape(1, c), beta.reshape(1, c), acc)


def _final_body(xc_ref, xa_ref, pol_ref, wf_ref, bf_ref, o_ref):
    p = pol_ref[...]
    e = jnp.exp(p - jnp.max(p))
    pp = e / jnp.sum(e)
    w_cat = pp[0:1, 0:1]
    w_adj = pp[0:1, 1:2] * jnp.float32(1.0 / POWER)
    x = w_cat * xc_ref[...] + w_adj * xa_ref[...]
    z = jnp.dot(x, wf_ref[...], preferred_element_type=jnp.float32)
    o_ref[...] = z + bf_ref[...]


def _final(xcat, xadj, policy, wf, bf):
    return pl.pallas_call(
        _final_body,
        grid=(8,),
        in_specs=[
            pl.BlockSpec((ROW_BLK, LAST_DIM), lambda i: (i, 0)),
            pl.BlockSpec((ROW_BLK, LAST_DIM), lambda i: (i, 0)),
            pl.BlockSpec((1, 2), lambda i: (0, 0)),
            pl.BlockSpec((LAST_DIM, OUT_C), lambda i: (0, 0)),
            pl.BlockSpec((1, OUT_C), lambda i: (0, 0)),
        ],
        out_specs=pl.BlockSpec((ROW_BLK, OUT_C), lambda i: (i, 0)),
        out_shape=jax.ShapeDtypeStruct((NPAD, OUT_C), jnp.float32),
    )(xcat, xadj, policy.reshape(1, 2), wf, bf.reshape(1, OUT_C))


def kernel(node_feat, row1, col1, val1, row2, col2, val2, W1, b1,
           bn1_gamma, bn1_beta, strc_W, strc_gamma, strc_beta, policy,
           Wf, bf):
    c1p, r1p, v1p, o1p = _prep_edges(row1, col1, val1)
    c2p, r2p, v2p, o2p = _prep_edges(row2, col2, val2)

    pad_rows = NPAD - N
    x0 = _embed(node_feat, W1, b1)
    x0p = jnp.concatenate(
        [x0, jnp.zeros((pad_rows, HID), jnp.float32)], axis=0)

    e1 = (c1p, r1p, v1p, o1p)
    e2 = (c2p, r2p, v2p, o2p)

    y1 = _conv_spmm(HID, x0p, e1, e2)
    x1 = _bn_apply(y1, _stats(y1), bn1_gamma, bn1_beta)

    x2 = _conv_spmm(2 * HID, x1, e1, e2)

    xcat = jnp.concatenate([x0p, x1, x2], axis=1)

    w = jnp.concatenate(
        [strc_W, jnp.zeros((pad_rows, LAST_DIM), jnp.float32)], axis=0)
    xadj = jnp.zeros((NPAD, LAST_DIM), jnp.float32)
    for i in range(POWER):
        t = _strc_spmm(w, e1)
        w, xadj = _bn_apply_acc(t, _stats(t), strc_gamma[i], strc_beta[i],
                                xadj)

    out = _final(xcat, xadj, policy, Wf, bf)
    return out[:N]


# speedup vs baseline: 16.1006x; 1.0021x over previous
import functools

import jax
import jax.numpy as jnp
from jax import lax
from jax.experimental import pallas as pl
from jax.experimental.pallas import tpu as pltpu
from jax.experimental.pallas import tpu_sc as plsc

N = 10000
IN_C = 128
HID = 64
OUT_C = 64
LAST_DIM = 448
POWER = 4

GROUP = 160
NGROUPS = 64
NPAD = GROUP * NGROUPS
NWORKERS = 32
ROW_BLK = 1280


EC = 128


def _lane_bcast(vec, t):
    return lax.gather(
        vec, jnp.full((16, 1), t, jnp.int32),
        lax.GatherDimensionNumbers(offset_dims=(), collapsed_slice_dims=(0,),
                                   start_index_map=(0,)),
        (1,), mode=lax.GatherScatterMode.PROMISE_IN_BOUNDS)


def _spmm_body(x_sh, colh, rowh, valh, offsv, out_hbm, colv, rowv, valv,
               gbuf, acc, sems, gsems, wid, c_p, c_out_off):
    nj = c_p // 16
    zero16 = jnp.zeros((16,), jnp.float32)

    for sub in range(2):
        g = wid * 2 + sub
        r0 = pl.multiple_of(g * GROUP, 8)
        off16 = offsv[pl.ds(g, 16)]
        ge0 = off16[0]
        ge1 = off16[1]
        base0 = (ge0 // 16) * 16
        nch = (ge1 - base0 + (EC - 1)) // EC
        nch2 = (nch + 1) // 2

        def zrow(r, _):
            for j in range(nj):
                acc[r, pl.ds(j * 16, 16)] = zero16
            return 0
        lax.fori_loop(0, GROUP, zrow, 0)

        def stage_col(k, s):
            b = pl.multiple_of(base0 + k * EC, 16)
            pltpu.async_copy(colh.at[pl.ds(b, EC)],
                             colv.at[pl.ds(s * EC, EC)], sems[s])

        def wait_col(s):
            pltpu.make_async_copy(colh.at[pl.ds(0, EC)],
                                  colv.at[pl.ds(s * EC, EC)],
                                  sems[s]).wait()

        def stage_vr(k, s):
            b = pl.multiple_of(base0 + k * EC, 16)
            sem = sems[2 + s]
            pltpu.async_copy(rowh.at[pl.ds(b, EC)],
                             rowv.at[pl.ds(s * EC, EC)], sem)
            pltpu.async_copy(valh.at[pl.ds(b, EC)],
                             valv.at[pl.ds(s * EC, EC)], sem)

        def wait_vr(s):
            sem = sems[2 + s]
            for _ in range(2):
                pltpu.make_async_copy(rowh.at[pl.ds(0, EC)],
                                      rowv.at[pl.ds(s * EC, EC)],
                                      sem).wait()

        def fire_gather(s):
            pltpu.async_copy(x_sh.at[colv.at[pl.ds(s * EC, EC)]],
                             gbuf.at[pl.ds(s * EC, EC)], gsems)

        def wait_gather(s):
            pltpu.make_async_copy(x_sh.at[colv.at[pl.ds(s * EC, EC)]],
                                  gbuf.at[pl.ds(s * EC, EC)], gsems).wait()

        stage_col(0, 0)
        stage_col(1, 1)
        stage_vr(0, 0)
        wait_col(0)
        fire_gather(0)

        def chunk_s(kk, s):
            k = 2 * kk + s
            soff = s * EC
            b = pl.multiple_of(base0 + k * EC, 16)
            wait_gather(s)
            wait_col(1 - s)
            fire_gather(1 - s)
            stage_col(k + 2, s)
            stage_vr(k + 1, 1 - s)
            wait_vr(s)

            def edge16(q, _):
                e0 = soff + q * 16
                eidx = (b - soff + e0) + lax.iota(jnp.int32, 16)
                ok = (eidx >= ge0) & (eidx < ge1)
                v16 = jnp.where(ok, valv[pl.ds(e0, 16)], 0.0)
                ro16 = jnp.clip(rowv[pl.ds(e0, 16)] - r0, 0, GROUP - 1)
                single = ro16[0] == ro16[15]

                @pl.when(single)
                def _():
                    vb = _lane_bcast(v16, 0)
                    rg = [vb * gbuf[e0, pl.ds(j * 16, 16)]
                          for j in range(nj)]
                    for t in range(1, 16):
                        vb = _lane_bcast(v16, t)
                        for j in range(nj):
                            rg[j] = rg[j] + vb * gbuf[e0 + t,
                                                      pl.ds(j * 16, 16)]
                    for j in range(nj):
                        plsc.addupdate(acc.at[ro16[0], pl.ds(j * 16, 16)],
                                       rg[j])

                @pl.when(jnp.logical_not(single))
                def _():
                    prow = ro16[0]
                    rg = [None] * nj
                    for t in range(16):
                        ro = ro16[t]
                        vb = _lane_bcast(v16, t)
                        if t == 0:
                            for j in range(nj):
                                rg[j] = vb * gbuf[e0, pl.ds(j * 16, 16)]
                            continue
                        flush = ro != prow

                        @pl.when(flush)
                        def _(prow=prow, rg=tuple(rg)):
                            for j in range(nj):
                                plsc.addupdate(
                                    acc.at[prow, pl.ds(j * 16, 16)], rg[j])
                        for j in range(nj):
                            contrib = vb * gbuf[e0 + t, pl.ds(j * 16, 16)]
                            rg[j] = jnp.where(flush, contrib,
                                              rg[j] + contrib)
                        prow = ro
                    for j in range(nj):
                        plsc.addupdate(acc.at[prow, pl.ds(j * 16, 16)],
                                       rg[j])
                return 0
            lax.fori_loop(0, EC // 16, edge16, 0)

        def chunk_pair(kk, _):
            chunk_s(kk, 0)
            chunk_s(kk, 1)
            return 0

        lax.fori_loop(0, nch2, chunk_pair, 0)
        wait_gather(0)
        wait_col(1)
        wait_vr(0)

        pltpu.sync_copy(
            acc, out_hbm.at[pl.ds(r0, GROUP), pl.ds(c_out_off, c_p)])


@functools.lru_cache(maxsize=None)
def _make_spmm_kernel(c_src: int, p_w: int, nsets: int):
    mesh = plsc.VectorSubcoreMesh(core_axis_name="c", subcore_axis_name="s")
    npan = c_src // p_w

    @functools.partial(
        pl.kernel,
        mesh=mesh,
        compiler_params=pltpu.CompilerParams(use_tc_tiling_on_sc=False),
        out_type=jax.ShapeDtypeStruct((NPAD, nsets * c_src), jnp.float32),
        scratch_types=[
            pltpu.VMEM_SHARED((NPAD, p_w), jnp.float32),
            pltpu.VMEM((2 * EC,), jnp.int32),
            pltpu.VMEM((2 * EC,), jnp.int32),
            pltpu.VMEM((2 * EC,), jnp.float32),
            pltpu.VMEM((2 * EC, p_w), jnp.float32),
            pltpu.VMEM((GROUP, p_w), jnp.float32),
        ] + [pltpu.VMEM((80,), jnp.int32)] * nsets + [
            pltpu.SemaphoreType.DMA,
            pltpu.SemaphoreType.DMA,
            pltpu.SemaphoreType.DMA,
            pltpu.SemaphoreType.DMA,
            pltpu.SemaphoreType.DMA,
            pltpu.SemaphoreType.DMA,
        ],
    )
    def spmm(*refs):
        x_hbm = refs[0]
        esets = [refs[1 + 4 * i:5 + 4 * i] for i in range(nsets)]
        out_hbm = refs[1 + 4 * nsets]
        sc = 2 + 4 * nsets
        x_sh, colv, rowv, valv, gbuf, acc = refs[sc:sc + 6]
        offsv = refs[sc + 6:sc + 6 + nsets]
        s0, s1, s2, s3, sg, sx = refs[sc + 6 + nsets:sc + 12 + nsets]
        sems = (s0, s1, s2, s3)

        sid = lax.axis_index("s")
        wid = sid * 2 + lax.axis_index("c")
        for i in range(nsets):
            pltpu.sync_copy(esets[i][3], offsv[i])
        for p in range(npan):
            plsc.subcore_barrier()

            @pl.when(sid == 0)
            def _(p=p):
                pltpu.async_copy(
                    x_hbm.at[:, pl.ds(p * p_w, p_w)], x_sh, sx).wait()
            plsc.subcore_barrier()
            for i in range(nsets):
                ch, rh, vh, _oh = esets[i]
                _spmm_body(x_sh, ch, rh, vh, offsv[i], out_hbm, colv, rowv,
                           valv, gbuf, acc, sems, sg, wid, p_w,
                           i * c_src + p * p_w)

    return spmm


def _conv_spmm(c_dim, x, e1, e2):
    return _make_spmm_kernel(c_dim, 64, 2)(x, *e1, *e2)


def _strc_spmm(x, e1):
    return _make_spmm_kernel(LAST_DIM, 112, 1)(x, *e1)


def _prep_edges(row, col, val):
    e = row.shape[0]
    pe = ((e + 1023) // 128) * 128
    pad = pe - e
    col32 = col.astype(jnp.int32)
    row32 = row.astype(jnp.int32)
    fill = (jnp.arange(pad, dtype=jnp.int32) * 97) % N
    colp = jnp.concatenate([col32, fill])
    rowp = jnp.concatenate([row32, jnp.zeros((pad,), jnp.int32)])
    valp = jnp.concatenate([val, jnp.zeros((pad,), jnp.float32)])
    bounds = jnp.arange(NGROUPS + 1, dtype=jnp.int32) * GROUP
    offs = jnp.searchsorted(row32, bounds).astype(jnp.int32)
    offsp = jnp.concatenate([offs, jnp.full((80 - (NGROUPS + 1),), e, jnp.int32)])
    return colp, rowp, valp, offsp


def _embed_body(nf_ref, w_ref, b_ref, o_ref):
    z = jnp.dot(nf_ref[...], w_ref[...], preferred_element_type=jnp.float32)
    o_ref[...] = jnp.maximum(z + b_ref[...], 0.0)


def _embed(node_feat, w1, b1):
    return pl.pallas_call(
        _embed_body,
        grid=(25,),
        in_specs=[
            pl.BlockSpec((400, IN_C), lambda i: (i, 0)),
            pl.BlockSpec((IN_C, HID), lambda i: (0, 0)),
            pl.BlockSpec((1, HID), lambda i: (0, 0)),
        ],
        out_specs=pl.BlockSpec((400, HID), lambda i: (i, 0)),
        out_shape=jax.ShapeDtypeStruct((N, HID), jnp.float32),
    )(node_feat, w1, b1.reshape(1, HID))


def _stats_body(y_ref, o_ref):
    i = pl.program_id(0)

    @pl.when(i == 0)
    def _():
        o_ref[...] = jnp.zeros_like(o_ref)

    blk = y_ref[...]
    s1 = jnp.sum(blk, axis=0, keepdims=True)
    s2 = jnp.sum(blk * blk, axis=0, keepdims=True)
    c = blk.shape[1]
    upd = jnp.concatenate([s1, s2, jnp.zeros((6, c), jnp.float32)], axis=0)
    o_ref[...] = o_ref[...] + upd


def _stats(y):
    c = y.shape[1]
    return pl.pallas_call(
        _stats_body,
        grid=(8,),
        in_specs=[pl.BlockSpec((ROW_BLK, c), lambda i: (i, 0))],
        out_specs=pl.BlockSpec((8, c), lambda i: (0, 0)),
        out_shape=jax.ShapeDtypeStruct((8, c), jnp.float32),
    )(y)


def _bn_body(y_ref, st_ref, g_ref, b_ref, o_ref):
    inv_n = jnp.float32(1.0 / N)
    mu = st_ref[0:1, :] * inv_n
    var = st_ref[1:2, :] * inv_n - mu * mu
    scale = lax.rsqrt(var + 1e-5) * g_ref[...]
    o_ref[...] = (y_ref[...] - mu) * scale + b_ref[...]


def _bn_apply(y, st, gamma, beta):
    c = y.shape[1]
    return pl.pallas_call(
        _bn_body,
        grid=(8,),
        in_specs=[
            pl.BlockSpec((ROW_BLK, c), lambda i: (i, 0)),
            pl.BlockSpec((8, c), lambda i: (0, 0)),
            pl.BlockSpec((1, c), lambda i: (0, 0)),
            pl.BlockSpec((1, c), lambda i: (0, 0)),
        ],
        out_specs=pl.BlockSpec((ROW_BLK, c), lambda i: (i, 0)),
        out_shape=jax.ShapeDtypeStruct((NPAD, c), jnp.float32),
    )(y, st, gamma.reshape(1, c), beta.reshape(1, c))


def _bn_acc_body(y_ref, st_ref, g_ref, b_ref, acc_ref, o_ref, accout_ref):
    inv_n = jnp.float32(1.0 / N)
    mu = st_ref[0:1, :] * inv_n
    var = st_ref[1:2, :] * inv_n - mu * mu
    scale = lax.rsqrt(var + 1e-5) * g_ref[...]
    z = (y_ref[...] - mu) * scale + b_ref[...]
    o_ref[...] = z
    accout_ref[...] = acc_ref[...] + z


def _bn_apply_acc(y, st, gamma, beta, acc):
    c = y.shape[1]
    return pl.pallas_call(
        _bn_acc_body,
        grid=(8,),
        in_specs=[
            pl.BlockSpec((ROW_BLK, c), lambda i: (i, 0)),
            pl.BlockSpec((8, c), lambda i: (0, 0)),
            pl.BlockSpec((1, c), lambda i: (0, 0)),
            pl.BlockSpec((1, c), lambda i: (0, 0)),
            pl.BlockSpec((ROW_BLK, c), lambda i: (i, 0)),
        ],
        out_specs=[
            pl.BlockSpec((ROW_BLK, c), lambda i: (i, 0)),
            pl.BlockSpec((ROW_BLK, c), lambda i: (i, 0)),
        ],
        out_shape=[
            jax.ShapeDtypeStruct((NPAD, c), jnp.float32),
            jax.ShapeDtypeStruct((NPAD, c), jnp.float32),
        ],
        input_output_aliases={4: 1},
    )(y, st, gamma.reshape(1, c), beta.reshape(1, c), acc)


def _final_body(xc_ref, xa_ref, pol_ref, wf_ref, bf_ref, o_ref):
    p = pol_ref[...]
    e = jnp.exp(p - jnp.max(p))
    pp = e / jnp.sum(e)
    w_cat = pp[0:1, 0:1]
    w_adj = pp[0:1, 1:2] * jnp.float32(1.0 / POWER)
    x = w_cat * xc_ref[...] + w_adj * xa_ref[...]
    z = jnp.dot(x, wf_ref[...], preferred_element_type=jnp.float32)
    o_ref[...] = z + bf_ref[...]


def _final(xcat, xadj, policy, wf, bf):
    return pl.pallas_call(
        _final_body,
        grid=(8,),
        in_specs=[
            pl.BlockSpec((ROW_BLK, LAST_DIM), lambda i: (i, 0)),
            pl.BlockSpec((ROW_BLK, LAST_DIM), lambda i: (i, 0)),
            pl.BlockSpec((1, 2), lambda i: (0, 0)),
            pl.BlockSpec((LAST_DIM, OUT_C), lambda i: (0, 0)),
            pl.BlockSpec((1, OUT_C), lambda i: (0, 0)),
        ],
        out_specs=pl.BlockSpec((ROW_BLK, OUT_C), lambda i: (i, 0)),
        out_shape=jax.ShapeDtypeStruct((NPAD, OUT_C), jnp.float32),
    )(xcat, xadj, policy.reshape(1, 2), wf, bf.reshape(1, OUT_C))


def kernel(node_feat, row1, col1, val1, row2, col2, val2, W1, b1,
           bn1_gamma, bn1_beta, strc_W, strc_gamma, strc_beta, policy,
           Wf, bf):
    c1p, r1p, v1p, o1p = _prep_edges(row1, col1, val1)
    c2p, r2p, v2p, o2p = _prep_edges(row2, col2, val2)

    pad_rows = NPAD - N
    x0 = _embed(node_feat, W1, b1)
    x0p = jnp.concatenate(
        [x0, jnp.zeros((pad_rows, HID), jnp.float32)], axis=0)

    e1 = (c1p, r1p, v1p, o1p)
    e2 = (c2p, r2p, v2p, o2p)

    y1 = _conv_spmm(HID, x0p, e1, e2)
    x1 = _bn_apply(y1, _stats(y1), bn1_gamma, bn1_beta)

    x2 = _conv_spmm(2 * HID, x1, e1, e2)

    xcat = jnp.concatenate([x0p, x1, x2], axis=1)

    w = jnp.concatenate(
        [strc_W, jnp.zeros((pad_rows, LAST_DIM), jnp.float32)], axis=0)
    xadj = jnp.zeros((NPAD, LAST_DIM), jnp.float32)
    for i in range(POWER):
        t = _strc_spmm(w, e1)
        w, xadj = _bn_apply_acc(t, _stats(t), strc_gamma[i], strc_beta[i],
                                xadj)

    out = _final(xcat, xadj, policy, Wf, bf)
    return out[:N]
